# baseline (speedup 1.0000x reference)
import functools

import jax
import jax.numpy as jnp
from jax import lax
from jax.experimental import pallas as pl
from jax.experimental.pallas import tpu as pltpu
from jax.experimental.pallas import tpu_sc as plsc

VOCAB_N = 1000000
EMBED = 128
HIDDEN = 128
OUT_N = 91
BATCH_N = 16384
CTX_N = 6

NC, NS = 2, 16
NW = NC * NS

NSEG = 4
BSEG = BATCH_N // NSEG
SEG_ROWS = BSEG * CTX_N
ROWS_PER_W = SEG_ROWS // NW
CHUNK = 256
NCHUNK = ROWS_PER_W // CHUNK
NBUF = 3


def _make_gather():
    mesh = plsc.VectorSubcoreMesh(core_axis_name="c", subcore_axis_name="s")

    @functools.partial(
        pl.kernel,
        mesh=mesh,
        out_type=jax.ShapeDtypeStruct((SEG_ROWS, EMBED), jnp.float32),
        scratch_types=(
            [pltpu.VMEM((ROWS_PER_W,), jnp.int32)]
            + [pltpu.VMEM((CHUNK, EMBED), jnp.float32)] * NBUF
            + [pltpu.SemaphoreType.DMA] * (2 * NBUF)
        ),
    )
    def gather(table_hbm, idx_hbm, out_hbm, idx_v, *scr):
        bufs = scr[:NBUF]
        gsem = scr[NBUF:2 * NBUF]
        wsem = scr[2 * NBUF:]
        wid = lax.axis_index("s") * NC + lax.axis_index("c")
        base = wid * ROWS_PER_W
        pltpu.sync_copy(idx_hbm.at[pl.ds(base, ROWS_PER_W)], idx_v)

        def fire_gather(c):
            pltpu.async_copy(
                table_hbm.at[idx_v.at[pl.ds(c * CHUNK, CHUNK)]],
                bufs[c % NBUF], gsem[c % NBUF])

        for c in range(min(2, NCHUNK)):
            fire_gather(c)
        for c in range(NCHUNK):
            b = c % NBUF
            pltpu.make_async_copy(
                table_hbm.at[idx_v.at[pl.ds(c * CHUNK, CHUNK)]],
                bufs[b], gsem[b]).wait()
            dst = out_hbm.at[pl.ds(base + c * CHUNK, CHUNK)]
            pltpu.async_copy(bufs[b], dst, wsem[b])
            n = c + 2
            if n < NCHUNK:
                nb = n % NBUF
                if n >= NBUF:
                    pltpu.make_async_copy(
                        bufs[nb],
                        out_hbm.at[pl.ds(base + (n - NBUF) * CHUNK, CHUNK)],
                        wsem[nb]).wait()
                fire_gather(n)
        for c in range(max(0, NCHUNK - NBUF), NCHUNK):
            b = c % NBUF
            pltpu.make_async_copy(
                bufs[b], out_hbm.at[pl.ds(base + c * CHUNK, CHUNK)],
                wsem[b]).wait()

    return gather


_gather = _make_gather()


def _mlp_body(x_ref, w1_ref, b1_ref, w2t_ref, b2t_ref, *rest):
    out_ref = rest[-1]
    h = b1_ref[...]
    for c in range(CTX_N):
        h = h + jnp.dot(x_ref[c], w1_ref[c],
                        preferred_element_type=jnp.float32)
    h = jnp.maximum(h, 0.0)
    out_t = jax.lax.dot_general(
        w2t_ref[...], h, (((1,), (1,)), ((), ())),
        preferred_element_type=jnp.float32)
    out_ref[...] = out_t + b2t_ref[...]


BM = 1024


def _mlp_seg(s, x3, W1r, b1, W2t, b2t, acc=None):
    grid = (BSEG // BM,)
    base = s * (BSEG // BM)
    in_specs = [
        pl.BlockSpec((CTX_N, BM, EMBED), lambda i: (0, i, 0)),
        pl.BlockSpec((CTX_N, EMBED, HIDDEN), lambda i: (0, 0, 0)),
        pl.BlockSpec((1, HIDDEN), lambda i: (0, 0)),
        pl.BlockSpec((OUT_N, HIDDEN), lambda i: (0, 0)),
        pl.BlockSpec((OUT_N, 1), lambda i: (0, 0)),
    ]
    args = [x3, W1r, b1, W2t, b2t]
    aliases = {}
    if acc is not None:
        in_specs.append(pl.BlockSpec(memory_space=pl.ANY))
        args.append(acc)
        aliases = {5: 0}
    return pl.pallas_call(
        _mlp_body,
        grid=grid,
        in_specs=in_specs,
        out_specs=pl.BlockSpec((OUT_N, BM), lambda i: (0, base + i)),
        out_shape=jax.ShapeDtypeStruct((OUT_N, BATCH_N), jnp.float32),
        input_output_aliases=aliases,
    )(*args)


def kernel(inputs, table, W1, b1, W2, b2):
    idx_t = inputs.T
    W1r = W1.reshape(CTX_N, EMBED, HIDDEN)
    b1r = b1.reshape(1, HIDDEN)
    W2t = W2.T
    b2t = b2.reshape(OUT_N, 1)
    logits_t = None
    for s in range(NSEG):
        idx_s = idx_t[:, s * BSEG:(s + 1) * BSEG].reshape(-1)
        embeds = _gather(table, idx_s)
        x3 = embeds.reshape(CTX_N, BSEG, EMBED)
        logits_t = _mlp_seg(s, x3, W1r, b1r, W2t, b2t, logits_t)
    return logits_t.T


# speedup vs baseline: 1.0362x; 1.0362x over previous
import functools

import jax
import jax.numpy as jnp
from jax import lax
from jax.experimental import pallas as pl
from jax.experimental.pallas import tpu as pltpu
from jax.experimental.pallas import tpu_sc as plsc

VOCAB_N = 1000000
EMBED = 128
HIDDEN = 128
OUT_N = 91
BATCH_N = 16384
CTX_N = 6

NC, NS = 2, 16
NW = NC * NS

NSEG = 2
BSEG = BATCH_N // NSEG
SEG_ROWS = BSEG * CTX_N
ROWS_PER_W = SEG_ROWS // NW
CHUNK = 256
NCHUNK = ROWS_PER_W // CHUNK
NBUF = 3


def _make_gather():
    mesh = plsc.VectorSubcoreMesh(core_axis_name="c", subcore_axis_name="s")

    @functools.partial(
        pl.kernel,
        mesh=mesh,
        out_type=jax.ShapeDtypeStruct((SEG_ROWS, EMBED), jnp.float32),
        scratch_types=(
            [pltpu.VMEM((ROWS_PER_W,), jnp.int32)]
            + [pltpu.VMEM((CHUNK, EMBED), jnp.float32)] * NBUF
            + [pltpu.SemaphoreType.DMA] * (2 * NBUF)
        ),
    )
    def gather(table_hbm, idx_hbm, out_hbm, idx_v, *scr):
        bufs = scr[:NBUF]
        gsem = scr[NBUF:2 * NBUF]
        wsem = scr[2 * NBUF:]
        wid = lax.axis_index("s") * NC + lax.axis_index("c")
        base = wid * ROWS_PER_W
        pltpu.sync_copy(idx_hbm.at[pl.ds(base, ROWS_PER_W)], idx_v)

        def fire_gather(c):
            pltpu.async_copy(
                table_hbm.at[idx_v.at[pl.ds(c * CHUNK, CHUNK)]],
                bufs[c % NBUF], gsem[c % NBUF])

        for c in range(min(2, NCHUNK)):
            fire_gather(c)
        for c in range(NCHUNK):
            b = c % NBUF
            pltpu.make_async_copy(
                table_hbm.at[idx_v.at[pl.ds(c * CHUNK, CHUNK)]],
                bufs[b], gsem[b]).wait()
            dst = out_hbm.at[pl.ds(base + c * CHUNK, CHUNK)]
            pltpu.async_copy(bufs[b], dst, wsem[b])
            n = c + 2
            if n < NCHUNK:
                nb = n % NBUF
                if n >= NBUF:
                    pltpu.make_async_copy(
                        bufs[nb],
                        out_hbm.at[pl.ds(base + (n - NBUF) * CHUNK, CHUNK)],
                        wsem[nb]).wait()
                fire_gather(n)
        for c in range(max(0, NCHUNK - NBUF), NCHUNK):
            b = c % NBUF
            pltpu.make_async_copy(
                bufs[b], out_hbm.at[pl.ds(base + c * CHUNK, CHUNK)],
                wsem[b]).wait()

    return gather


_gather = _make_gather()


def _mlp_body(x_ref, w1_ref, b1_ref, w2t_ref, b2t_ref, *rest):
    out_ref = rest[-1]
    h = b1_ref[...]
    for c in range(CTX_N):
        h = h + jnp.dot(x_ref[c], w1_ref[c],
                        preferred_element_type=jnp.float32)
    h = jnp.maximum(h, 0.0)
    out_t = jax.lax.dot_general(
        w2t_ref[...], h, (((1,), (1,)), ((), ())),
        preferred_element_type=jnp.float32)
    out_ref[...] = out_t + b2t_ref[...]


BM = 1024


def _mlp_seg(s, x3, W1r, b1, W2t, b2t, acc=None):
    grid = (BSEG // BM,)
    base = s * (BSEG // BM)
    in_specs = [
        pl.BlockSpec((CTX_N, BM, EMBED), lambda i: (0, i, 0)),
        pl.BlockSpec((CTX_N, EMBED, HIDDEN), lambda i: (0, 0, 0)),
        pl.BlockSpec((1, HIDDEN), lambda i: (0, 0)),
        pl.BlockSpec((OUT_N, HIDDEN), lambda i: (0, 0)),
        pl.BlockSpec((OUT_N, 1), lambda i: (0, 0)),
    ]
    args = [x3, W1r, b1, W2t, b2t]
    aliases = {}
    if acc is not None:
        in_specs.append(pl.BlockSpec(memory_space=pl.ANY))
        args.append(acc)
        aliases = {5: 0}
    return pl.pallas_call(
        _mlp_body,
        grid=grid,
        in_specs=in_specs,
        out_specs=pl.BlockSpec((OUT_N, BM), lambda i: (0, base + i)),
        out_shape=jax.ShapeDtypeStruct((OUT_N, BATCH_N), jnp.float32),
        input_output_aliases=aliases,
    )(*args)


def kernel(inputs, table, W1, b1, W2, b2):
    idx_t = inputs.T
    W1r = W1.reshape(CTX_N, EMBED, HIDDEN)
    b1r = b1.reshape(1, HIDDEN)
    W2t = W2.T
    b2t = b2.reshape(OUT_N, 1)
    logits_t = None
    for s in range(NSEG):
        idx_s = idx_t[:, s * BSEG:(s + 1) * BSEG].reshape(-1)
        embeds = _gather(table, idx_s)
        x3 = embeds.reshape(CTX_N, BSEG, EMBED)
        logits_t = _mlp_seg(s, x3, W1r, b1r, W2t, b2t, logits_t)
    return logits_t.T


# speedup vs baseline: 1.0677x; 1.0304x over previous
import functools

import jax
import jax.numpy as jnp
from jax import lax
from jax.experimental import pallas as pl
from jax.experimental.pallas import tpu as pltpu
from jax.experimental.pallas import tpu_sc as plsc

VOCAB_N = 1000000
EMBED = 128
HIDDEN = 128
OUT_N = 91
BATCH_N = 16384
CTX_N = 6

NC, NS = 2, 16
NW = NC * NS

NSEG = 2
BSEG = BATCH_N // NSEG
SEG_ROWS = BSEG * CTX_N
ROWS_PER_W = SEG_ROWS // NW
CHUNK = 256
NCHUNK = ROWS_PER_W // CHUNK
NBUF = 3


def _make_gather():
    mesh = plsc.VectorSubcoreMesh(core_axis_name="c", subcore_axis_name="s")

    @functools.partial(
        pl.kernel,
        mesh=mesh,
        out_type=jax.ShapeDtypeStruct((SEG_ROWS, EMBED), jnp.float32),
        scratch_types=(
            [pltpu.VMEM((ROWS_PER_W,), jnp.int32)]
            + [pltpu.VMEM((CHUNK, EMBED), jnp.float32)] * NBUF
            + [pltpu.SemaphoreType.DMA] * (2 * NBUF)
        ),
    )
    def gather(table_hbm, idx_hbm, out_hbm, idx_v, *scr):
        bufs = scr[:NBUF]
        gsem = scr[NBUF:2 * NBUF]
        wsem = scr[2 * NBUF:]
        wid = lax.axis_index("s") * NC + lax.axis_index("c")
        base = wid * ROWS_PER_W
        pltpu.sync_copy(idx_hbm.at[pl.ds(base, ROWS_PER_W)], idx_v)

        def fire_gather(c):
            pltpu.async_copy(
                table_hbm.at[idx_v.at[pl.ds(c * CHUNK, CHUNK)]],
                bufs[c % NBUF], gsem[c % NBUF])

        for c in range(min(2, NCHUNK)):
            fire_gather(c)
        for c in range(NCHUNK):
            b = c % NBUF
            pltpu.make_async_copy(
                table_hbm.at[idx_v.at[pl.ds(c * CHUNK, CHUNK)]],
                bufs[b], gsem[b]).wait()
            dst = out_hbm.at[pl.ds(base + c * CHUNK, CHUNK)]
            pltpu.async_copy(bufs[b], dst, wsem[b])
            n = c + 2
            if n < NCHUNK:
                nb = n % NBUF
                if n >= NBUF:
                    pltpu.make_async_copy(
                        bufs[nb],
                        out_hbm.at[pl.ds(base + (n - NBUF) * CHUNK, CHUNK)],
                        wsem[nb]).wait()
                fire_gather(n)
        for c in range(max(0, NCHUNK - NBUF), NCHUNK):
            b = c % NBUF
            pltpu.make_async_copy(
                bufs[b], out_hbm.at[pl.ds(base + c * CHUNK, CHUNK)],
                wsem[b]).wait()

    return gather


_gather = _make_gather()


def _mlp_body(x_ref, w1_ref, b1_ref, w2t_ref, b2t_ref, *rest):
    out_ref = rest[-1]
    h = b1_ref[...]
    for c in range(CTX_N):
        h = h + jnp.dot(x_ref[c], w1_ref[c],
                        preferred_element_type=jnp.float32)
    h = jnp.maximum(h, 0.0)
    out_t = jax.lax.dot_general(
        w2t_ref[...], h, (((1,), (1,)), ((), ())),
        preferred_element_type=jnp.float32)
    out_ref[...] = out_t + b2t_ref[...]


BM = 2048


def _mlp_seg(s, x3, W1r, b1, W2t, b2t, acc=None):
    grid = (BSEG // BM,)
    base = s * (BSEG // BM)
    in_specs = [
        pl.BlockSpec((CTX_N, BM, EMBED), lambda i: (0, i, 0)),
        pl.BlockSpec((CTX_N, EMBED, HIDDEN), lambda i: (0, 0, 0)),
        pl.BlockSpec((1, HIDDEN), lambda i: (0, 0)),
        pl.BlockSpec((OUT_N, HIDDEN), lambda i: (0, 0)),
        pl.BlockSpec((OUT_N, 1), lambda i: (0, 0)),
    ]
    args = [x3, W1r, b1, W2t, b2t]
    aliases = {}
    if acc is not None:
        in_specs.append(pl.BlockSpec(memory_space=pl.ANY))
        args.append(acc)
        aliases = {5: 0}
    return pl.pallas_call(
        _mlp_body,
        grid=grid,
        in_specs=in_specs,
        out_specs=pl.BlockSpec((OUT_N, BM), lambda i: (0, base + i)),
        out_shape=jax.ShapeDtypeStruct((OUT_N, BATCH_N), jnp.float32),
        input_output_aliases=aliases,
    )(*args)


def kernel(inputs, table, W1, b1, W2, b2):
    idx_t = inputs.T
    W1r = W1.reshape(CTX_N, EMBED, HIDDEN)
    b1r = b1.reshape(1, HIDDEN)
    W2t = W2.T
    b2t = b2.reshape(OUT_N, 1)
    logits_t = None
    for s in range(NSEG):
        idx_s = idx_t[:, s * BSEG:(s + 1) * BSEG].reshape(-1)
        embeds = _gather(table, idx_s)
        x3 = embeds.reshape(CTX_N, BSEG, EMBED)
        logits_t = _mlp_seg(s, x3, W1r, b1r, W2t, b2t, logits_t)
    return logits_t.T


# speedup vs baseline: 1.0820x; 1.0133x over previous
import functools

import jax
import jax.numpy as jnp
from jax import lax
from jax.experimental import pallas as pl
from jax.experimental.pallas import tpu as pltpu
from jax.experimental.pallas import tpu_sc as plsc

VOCAB_N = 1000000
EMBED = 128
HIDDEN = 128
OUT_N = 91
BATCH_N = 16384
CTX_N = 6

NC, NS = 2, 16
NW = NC * NS

NSEG = 2
BSEG = BATCH_N // NSEG
SEG_ROWS = BSEG * CTX_N
ROWS_PER_W = SEG_ROWS // NW
CHUNK = 256
NCHUNK = ROWS_PER_W // CHUNK
NBUF = 3


def _make_gather():
    mesh = plsc.VectorSubcoreMesh(core_axis_name="c", subcore_axis_name="s")

    @functools.partial(
        pl.kernel,
        mesh=mesh,
        out_type=jax.ShapeDtypeStruct((SEG_ROWS, EMBED), jnp.float32),
        scratch_types=(
            [pltpu.VMEM((ROWS_PER_W,), jnp.int32)]
            + [pltpu.VMEM((CHUNK, EMBED), jnp.float32)] * NBUF
            + [pltpu.SemaphoreType.DMA] * (2 * NBUF)
        ),
    )
    def gather(table_hbm, idx_hbm, out_hbm, idx_v, *scr):
        bufs = scr[:NBUF]
        gsem = scr[NBUF:2 * NBUF]
        wsem = scr[2 * NBUF:]
        wid = lax.axis_index("s") * NC + lax.axis_index("c")
        base = wid * ROWS_PER_W
        pltpu.sync_copy(idx_hbm.at[pl.ds(base, ROWS_PER_W)], idx_v)

        def fire_gather(c):
            pltpu.async_copy(
                table_hbm.at[idx_v.at[pl.ds(c * CHUNK, CHUNK)]],
                bufs[c % NBUF], gsem[c % NBUF])

        for c in range(min(2, NCHUNK)):
            fire_gather(c)
        for c in range(NCHUNK):
            b = c % NBUF
            pltpu.make_async_copy(
                table_hbm.at[idx_v.at[pl.ds(c * CHUNK, CHUNK)]],
                bufs[b], gsem[b]).wait()
            dst = out_hbm.at[pl.ds(base + c * CHUNK, CHUNK)]
            pltpu.async_copy(bufs[b], dst, wsem[b])
            n = c + 2
            if n < NCHUNK:
                nb = n % NBUF
                if n >= NBUF:
                    pltpu.make_async_copy(
                        bufs[nb],
                        out_hbm.at[pl.ds(base + (n - NBUF) * CHUNK, CHUNK)],
                        wsem[nb]).wait()
                fire_gather(n)
        for c in range(max(0, NCHUNK - NBUF), NCHUNK):
            b = c % NBUF
            pltpu.make_async_copy(
                bufs[b], out_hbm.at[pl.ds(base + c * CHUNK, CHUNK)],
                wsem[b]).wait()

    return gather


_gather = _make_gather()


def _mlp_body(x_ref, w1_ref, b1_ref, w2t_ref, b2t_ref, *rest):
    out_ref = rest[-1]
    h = b1_ref[...]
    for c in range(CTX_N):
        h = h + jnp.dot(x_ref[c], w1_ref[c],
                        preferred_element_type=jnp.float32)
    h = jnp.maximum(h, 0.0)
    out_t = jax.lax.dot_general(
        w2t_ref[...], h, (((1,), (1,)), ((), ())),
        preferred_element_type=jnp.float32)
    out_ref[...] = out_t + b2t_ref[...]


BM = 4096


def _mlp_seg(s, x3, W1r, b1, W2t, b2t, acc=None):
    grid = (BSEG // BM,)
    base = s * (BSEG // BM)
    in_specs = [
        pl.BlockSpec((CTX_N, BM, EMBED), lambda i: (0, i, 0)),
        pl.BlockSpec((CTX_N, EMBED, HIDDEN), lambda i: (0, 0, 0)),
        pl.BlockSpec((1, HIDDEN), lambda i: (0, 0)),
        pl.BlockSpec((OUT_N, HIDDEN), lambda i: (0, 0)),
        pl.BlockSpec((OUT_N, 1), lambda i: (0, 0)),
    ]
    args = [x3, W1r, b1, W2t, b2t]
    aliases = {}
    if acc is not None:
        in_specs.append(pl.BlockSpec(memory_space=pl.ANY))
        args.append(acc)
        aliases = {5: 0}
    return pl.pallas_call(
        _mlp_body,
        grid=grid,
        in_specs=in_specs,
        out_specs=pl.BlockSpec((OUT_N, BM), lambda i: (0, base + i)),
        out_shape=jax.ShapeDtypeStruct((OUT_N, BATCH_N), jnp.float32),
        input_output_aliases=aliases,
    )(*args)


def kernel(inputs, table, W1, b1, W2, b2):
    idx_t = inputs.T
    W1r = W1.reshape(CTX_N, EMBED, HIDDEN)
    b1r = b1.reshape(1, HIDDEN)
    W2t = W2.T
    b2t = b2.reshape(OUT_N, 1)
    logits_t = None
    for s in range(NSEG):
        idx_s = idx_t[:, s * BSEG:(s + 1) * BSEG].reshape(-1)
        embeds = _gather(table, idx_s)
        x3 = embeds.reshape(CTX_N, BSEG, EMBED)
        logits_t = _mlp_seg(s, x3, W1r, b1r, W2t, b2t, logits_t)
    return logits_t.T


# speedup vs baseline: 1.1048x; 1.0211x over previous
import functools

import jax
import jax.numpy as jnp
from jax import lax
from jax.experimental import pallas as pl
from jax.experimental.pallas import tpu as pltpu
from jax.experimental.pallas import tpu_sc as plsc

VOCAB_N = 1000000
EMBED = 128
HIDDEN = 128
OUT_N = 91
BATCH_N = 16384
CTX_N = 6

NC, NS = 2, 16
NW = NC * NS

NSEG = 2
BSEG = BATCH_N // NSEG
SEG_ROWS = BSEG * CTX_N
ROWS_PER_W = SEG_ROWS // NW
CHUNK = 192
NCHUNK = ROWS_PER_W // CHUNK
NBUF = 5


def _make_gather():
    mesh = plsc.VectorSubcoreMesh(core_axis_name="c", subcore_axis_name="s")

    @functools.partial(
        pl.kernel,
        mesh=mesh,
        out_type=jax.ShapeDtypeStruct((SEG_ROWS, EMBED), jnp.float32),
        scratch_types=(
            [pltpu.VMEM((ROWS_PER_W,), jnp.int32)]
            + [pltpu.VMEM((CHUNK, EMBED), jnp.float32)] * NBUF
            + [pltpu.SemaphoreType.DMA] * (2 * NBUF)
        ),
    )
    def gather(table_hbm, idx_hbm, out_hbm, idx_v, *scr):
        bufs = scr[:NBUF]
        gsem = scr[NBUF:2 * NBUF]
        wsem = scr[2 * NBUF:]
        wid = lax.axis_index("s") * NC + lax.axis_index("c")
        base = wid * ROWS_PER_W
        pltpu.sync_copy(idx_hbm.at[pl.ds(base, ROWS_PER_W)], idx_v)

        def fire_gather(c):
            pltpu.async_copy(
                table_hbm.at[idx_v.at[pl.ds(c * CHUNK, CHUNK)]],
                bufs[c % NBUF], gsem[c % NBUF])

        for c in range(min(NBUF - 1, NCHUNK)):
            fire_gather(c)
        for c in range(NCHUNK):
            b = c % NBUF
            pltpu.make_async_copy(
                table_hbm.at[idx_v.at[pl.ds(c * CHUNK, CHUNK)]],
                bufs[b], gsem[b]).wait()
            dst = out_hbm.at[pl.ds(base + c * CHUNK, CHUNK)]
            pltpu.async_copy(bufs[b], dst, wsem[b])
            n = c + NBUF - 1
            if n < NCHUNK:
                nb = n % NBUF
                if n >= NBUF:
                    pltpu.make_async_copy(
                        bufs[nb],
                        out_hbm.at[pl.ds(base + (n - NBUF) * CHUNK, CHUNK)],
                        wsem[nb]).wait()
                fire_gather(n)
        for c in range(max(0, NCHUNK - NBUF), NCHUNK):
            b = c % NBUF
            pltpu.make_async_copy(
                bufs[b], out_hbm.at[pl.ds(base + c * CHUNK, CHUNK)],
                wsem[b]).wait()

    return gather


_gather = _make_gather()


def _mlp_body(x_ref, w1_ref, b1_ref, w2t_ref, b2t_ref, *rest):
    out_ref = rest[-1]
    h = b1_ref[...]
    for c in range(CTX_N):
        h = h + jnp.dot(x_ref[c], w1_ref[c],
                        preferred_element_type=jnp.float32)
    h = jnp.maximum(h, 0.0)
    out_t = jax.lax.dot_general(
        w2t_ref[...], h, (((1,), (1,)), ((), ())),
        preferred_element_type=jnp.float32)
    out_ref[...] = out_t + b2t_ref[...]


BM = 4096


def _mlp_seg(s, x3, W1r, b1, W2t, b2t, acc=None):
    grid = (BSEG // BM,)
    base = s * (BSEG // BM)
    in_specs = [
        pl.BlockSpec((CTX_N, BM, EMBED), lambda i: (0, i, 0)),
        pl.BlockSpec((CTX_N, EMBED, HIDDEN), lambda i: (0, 0, 0)),
        pl.BlockSpec((1, HIDDEN), lambda i: (0, 0)),
        pl.BlockSpec((OUT_N, HIDDEN), lambda i: (0, 0)),
        pl.BlockSpec((OUT_N, 1), lambda i: (0, 0)),
    ]
    args = [x3, W1r, b1, W2t, b2t]
    aliases = {}
    if acc is not None:
        in_specs.append(pl.BlockSpec(memory_space=pl.ANY))
        args.append(acc)
        aliases = {5: 0}
    return pl.pallas_call(
        _mlp_body,
        grid=grid,
        in_specs=in_specs,
        out_specs=pl.BlockSpec((OUT_N, BM), lambda i: (0, base + i)),
        out_shape=jax.ShapeDtypeStruct((OUT_N, BATCH_N), jnp.float32),
        input_output_aliases=aliases,
    )(*args)


def kernel(inputs, table, W1, b1, W2, b2):
    idx_t = inputs.T
    W1r = W1.reshape(CTX_N, EMBED, HIDDEN)
    b1r = b1.reshape(1, HIDDEN)
    W2t = W2.T
    b2t = b2.reshape(OUT_N, 1)
    logits_t = None
    for s in range(NSEG):
        idx_s = idx_t[:, s * BSEG:(s + 1) * BSEG].reshape(-1)
        embeds = _gather(table, idx_s)
        x3 = embeds.reshape(CTX_N, BSEG, EMBED)
        logits_t = _mlp_seg(s, x3, W1r, b1r, W2t, b2t, logits_t)
    return logits_t.T
